# Initial kernel scaffold; baseline (speedup 1.0000x reference)
import jax
import jax.numpy as jnp
from jax.experimental import pallas as pl


def kernel(X, Y, intercept):
    raise NotImplementedError("write your pallas kernel here")


# speedup vs baseline: 2.5170x; 2.5170x over previous
import functools

import jax
import jax.numpy as jnp
from jax import lax
from jax.experimental import pallas as pl

_TEMP = 50.0
_MAX_EFF_TEMP = 5000.0


def _fcm_body(x_ref, yq_ref, b_ref, choice_ref, v_ref):
    x = x_ref[...]
    yq = yq_ref[...]
    b = b_ref[...]
    scores = lax.dot_general(
        x, yq, (((1,), (1,)), ((), ())),
        preferred_element_type=jnp.float32,
    ) + b
    m = jnp.max(scores, axis=1, keepdims=True)
    mn = jnp.min(scores, axis=1, keepdims=True)
    span = jnp.maximum(m - mn, 1e-3)
    eff = jnp.clip(_TEMP / span, _TEMP, _MAX_EFF_TEMP)
    e = jnp.exp((scores - m) * eff)
    denom = jnp.sum(e, axis=1, keepdims=True)
    w = e / denom
    v_ref[...] = jnp.sum(w * scores, axis=1, keepdims=True)
    choice_ref[...] = jnp.dot(w, yq, preferred_element_type=jnp.float32)


@functools.partial(jax.jit, static_argnames=("block_s",))
def _fcm(X, Y, intercept, block_s=256):
    S, d = X.shape
    K = Y.shape[1]
    yq = Y[0]
    grid = (S // block_s,)
    choice, v = pl.pallas_call(
        _fcm_body,
        grid=grid,
        in_specs=[
            pl.BlockSpec((block_s, d), lambda i: (i, 0)),
            pl.BlockSpec((K, d), lambda i: (0, 0)),
            pl.BlockSpec((1, K), lambda i: (0, 0)),
        ],
        out_specs=[
            pl.BlockSpec((block_s, d), lambda i: (i, 0)),
            pl.BlockSpec((block_s, 1), lambda i: (i, 0)),
        ],
        out_shape=[
            jax.ShapeDtypeStruct((S, d), jnp.float32),
            jax.ShapeDtypeStruct((S, 1), jnp.float32),
        ],
    )(X, yq, intercept)
    return choice, v[:, 0]


def kernel(X, Y, intercept):
    return _fcm(X, Y, intercept)


# speedup vs baseline: 2.6017x; 1.0337x over previous
import functools

import jax
import jax.numpy as jnp
from jax import lax
from jax.experimental import pallas as pl

_TEMP = 50.0
_MAX_EFF_TEMP = 5000.0


def _fcm_body(x_ref, yq_ref, b_ref, choice_ref, v_ref):
    x = x_ref[...]
    yq = yq_ref[...]
    b = b_ref[...]
    scores = lax.dot_general(
        x, yq, (((1,), (1,)), ((), ())),
        preferred_element_type=jnp.float32,
    ) + b
    m = jnp.max(scores, axis=1, keepdims=True)
    mn = jnp.min(scores, axis=1, keepdims=True)
    span = jnp.maximum(m - mn, 1e-3)
    eff = jnp.clip(_TEMP / span, _TEMP, _MAX_EFF_TEMP)
    e = jnp.exp(scores * eff - m * eff)
    denom = jnp.sum(e, axis=1, keepdims=True)
    inv = 1.0 / denom
    v_ref[...] = jnp.sum(e * scores, axis=1, keepdims=True) * inv
    choice_ref[...] = jnp.dot(e, yq, preferred_element_type=jnp.float32) * inv


@functools.partial(jax.jit, static_argnames=("block_s",))
def _fcm(X, Y, intercept, block_s=256):
    S, d = X.shape
    K = Y.shape[1]
    yq = Y[0]
    grid = (S // block_s,)
    choice, v = pl.pallas_call(
        _fcm_body,
        grid=grid,
        in_specs=[
            pl.BlockSpec((block_s, d), lambda i: (i, 0)),
            pl.BlockSpec((K, d), lambda i: (0, 0)),
            pl.BlockSpec((1, K), lambda i: (0, 0)),
        ],
        out_specs=[
            pl.BlockSpec((block_s, d), lambda i: (i, 0)),
            pl.BlockSpec((block_s, 1), lambda i: (i, 0)),
        ],
        out_shape=[
            jax.ShapeDtypeStruct((S, d), jnp.float32),
            jax.ShapeDtypeStruct((S, 1), jnp.float32),
        ],
    )(X, yq, intercept)
    return choice, v[:, 0]


def kernel(X, Y, intercept):
    return _fcm(X, Y, intercept)


# speedup vs baseline: 2.6810x; 1.0305x over previous
import functools

import jax
import jax.numpy as jnp
from jax import lax
from jax.experimental import pallas as pl
from jax.experimental.pallas import tpu as pltpu

_TEMP = 50.0
_MAX_EFF_TEMP = 5000.0


def _fcm_body(x_ref, yq_ref, b_ref, choice_ref, v_ref, s_ref, *, nt):
    bs, d = x_ref.shape
    k = yq_ref.shape[0]
    tk = k // nt
    x = x_ref[...]

    m = None
    mn = None
    for t in range(nt):
        yq_t = yq_ref[pl.ds(t * tk, tk), :]
        s_t = lax.dot_general(
            x, yq_t, (((1,), (1,)), ((), ())),
            preferred_element_type=jnp.float32,
        ) + b_ref[:, pl.ds(t * tk, tk)]
        s_ref[:, pl.ds(t * tk, tk)] = s_t
        m_t = jnp.max(s_t, axis=1, keepdims=True)
        mn_t = jnp.min(s_t, axis=1, keepdims=True)
        m = m_t if m is None else jnp.maximum(m, m_t)
        mn = mn_t if mn is None else jnp.minimum(mn, mn_t)

    span = jnp.maximum(m - mn, 1e-3)
    eff = jnp.clip(_TEMP / span, _TEMP, _MAX_EFF_TEMP)
    c = m * eff

    denom = jnp.zeros((bs, 1), jnp.float32)
    ve = jnp.zeros((bs, 1), jnp.float32)
    acc = jnp.zeros((bs, d), jnp.float32)
    for t in range(nt):
        s_t = s_ref[:, pl.ds(t * tk, tk)]
        e_t = jnp.exp(s_t * eff - c)
        denom = denom + jnp.sum(e_t, axis=1, keepdims=True)
        ve = ve + jnp.sum(e_t * s_t, axis=1, keepdims=True)
        acc = acc + jnp.dot(
            e_t, yq_ref[pl.ds(t * tk, tk), :],
            preferred_element_type=jnp.float32,
        )

    inv = 1.0 / denom
    v_ref[...] = ve * inv
    choice_ref[...] = acc * inv


@functools.partial(jax.jit, static_argnames=("block_s", "nt"))
def _fcm(X, Y, intercept, block_s=256, nt=8):
    S, d = X.shape
    K = Y.shape[1]
    yq = Y[0]
    grid = (S // block_s,)
    choice, v = pl.pallas_call(
        functools.partial(_fcm_body, nt=nt),
        grid=grid,
        in_specs=[
            pl.BlockSpec((block_s, d), lambda i: (i, 0)),
            pl.BlockSpec((K, d), lambda i: (0, 0)),
            pl.BlockSpec((1, K), lambda i: (0, 0)),
        ],
        out_specs=[
            pl.BlockSpec((block_s, d), lambda i: (i, 0)),
            pl.BlockSpec((block_s, 1), lambda i: (i, 0)),
        ],
        out_shape=[
            jax.ShapeDtypeStruct((S, d), jnp.float32),
            jax.ShapeDtypeStruct((S, 1), jnp.float32),
        ],
        scratch_shapes=[pltpu.VMEM((block_s, K), jnp.float32)],
    )(X, yq, intercept)
    return choice, v[:, 0]


def kernel(X, Y, intercept):
    return _fcm(X, Y, intercept)
